# baseline (speedup 1.0000x reference)
import jax
import jax.numpy as jnp
from jax import lax
from jax.experimental import pallas as pl
from jax.sharding import PartitionSpec as P

_BATCH = 16384
_BM = 4096


def _dot(a, b):
    return lax.dot_general(a, b, (((1,), (0,)), ((), ())),
                           preferred_element_type=jnp.float32)


def _mlp_body(f_ref, idx_ref, pe_ref, w1_ref, b1_ref, w2_ref, b2_ref,
              w3_ref, b3_ref, wd1_ref, bd1_ref, wd2_ref, bd2_ref,
              emb_ref, rec_ref):
    w1b = w1_ref[128:136, :]
    t = _dot(pe_ref[...].astype(jnp.bfloat16),
             w1b.astype(jnp.bfloat16)) + b1_ref[...]
    t = jnp.concatenate([t, jnp.zeros((2, 32), jnp.float32)], axis=0)
    w3d = jnp.concatenate(
        [w3_ref[...],
         _dot(w3_ref[...].astype(jnp.bfloat16),
              wd1_ref[...].astype(jnp.bfloat16))], axis=1)
    b3d = jnp.concatenate(
        [b3_ref[...],
         _dot(b3_ref[...].astype(jnp.bfloat16),
              wd1_ref[...].astype(jnp.bfloat16)) + bd1_ref[...]], axis=1)

    idxb = jnp.broadcast_to(idx_ref[...], (8, _BM))
    iota = lax.broadcasted_iota(jnp.int32, (8, _BM), 0)
    onehot_t = (idxb == iota).astype(jnp.bfloat16)
    g = lax.dot_general(onehot_t, t.astype(jnp.bfloat16),
                        (((0,), (0,)), ((), ())),
                        preferred_element_type=jnp.float32)

    h1 = _dot(f_ref[...].astype(jnp.bfloat16), w1_ref[:128, :].astype(jnp.bfloat16))
    h1 = jnp.maximum(h1 + g, 0.0).astype(jnp.bfloat16)
    h2 = _dot(h1, w2_ref[...].astype(jnp.bfloat16))
    h2 = jnp.maximum(h2 + b2_ref[...], 0.0).astype(jnp.bfloat16)
    ed = _dot(h2, w3d.astype(jnp.bfloat16)) + b3d
    emb_ref[...] = ed[:, :16]
    d = jnp.maximum(ed[:, 16:], 0.0).astype(jnp.bfloat16)
    rec_ref[...] = _dot(d, wd2_ref[...].astype(jnp.bfloat16)) + bd2_ref[...]


def _run_local(features, idx2d, pos_emb, W1, b1, W2, b2, W3, b3,
               Wd1, bd1, Wd2, bd2):
    batch = features.shape[0]
    nb = batch // _BM
    full = lambda shape: pl.BlockSpec(shape, lambda i: (0, 0))
    emb, rec = pl.pallas_call(
        _mlp_body,
        grid=(nb,),
        in_specs=[
            pl.BlockSpec((_BM, 128), lambda i: (i, 0)),
            pl.BlockSpec((1, _BM), lambda i: (0, i)),
            full((6, 8)),
            full((136, 32)),
            full((1, 32)),
            full((32, 16)),
            full((1, 16)),
            full((16, 16)),
            full((1, 16)),
            full((16, 16)),
            full((1, 16)),
            full((16, 128)),
            full((1, 128)),
        ],
        out_specs=[
            pl.BlockSpec((_BM, 16), lambda i: (i, 0)),
            pl.BlockSpec((_BM, 128), lambda i: (i, 0)),
        ],
        out_shape=[
            jax.ShapeDtypeStruct((batch, 16), jnp.float32),
            jax.ShapeDtypeStruct((batch, 128), jnp.float32),
        ],
    )(features, idx2d, pos_emb, W1, b1, W2, b2, W3, b3,
      Wd1, bd1, Wd2, bd2)
    return (emb, rec)


def kernel(features, position_idx, pos_emb, W1, b1, W2, b2, W3, b3,
           Wd1, bd1, Wd2, bd2):
    nd = 2 if jax.device_count() >= 2 else 1
    mesh = jax.make_mesh((nd,), ("b",))
    idx2d = position_idx.reshape(nd, _BATCH // nd)
    row = P("b", None)
    rep = P(None, None)
    f = jax.shard_map(
        _run_local,
        mesh=mesh,
        in_specs=(row, row) + (rep,) * 11,
        out_specs=(row, row),
        check_vma=False,
    )
    args = (features, idx2d, pos_emb, W1, b1[None, :], W2, b2[None, :],
            W3, b3[None, :], Wd1, bd1[None, :], Wd2, bd2[None, :])
    specs = (row, row) + (rep,) * 11
    args = tuple(
        jax.reshard(a, jax.sharding.NamedSharding(mesh, s))
        for a, s in zip(args, specs))
    emb, rec = f(*args)
    return (emb, rec)
